# Initial kernel scaffold; baseline (speedup 1.0000x reference)
import jax
import jax.numpy as jnp
from jax.experimental import pallas as pl


def kernel(x, w1, w2):
    raise NotImplementedError("write your pallas kernel here")


# speedup vs baseline: 1.0327x; 1.0327x over previous
import functools

import jax
import jax.numpy as jnp
from jax.experimental import pallas as pl
from jax.experimental.pallas import tpu as pltpu


def _se_body(x_ref, w1t_ref, w2t_ref, o_ref):
    y = jnp.sum(x_ref[...], axis=-1)
    h = jnp.dot(y, w1t_ref[...], preferred_element_type=jnp.float32)
    h = jnp.maximum(h, 0.0)
    logits = jnp.dot(h, w2t_ref[...], preferred_element_type=jnp.float32)

    m = jnp.max(logits, axis=-1, keepdims=True)
    e = jnp.exp(logits - m)
    o_ref[...] = e * pl.reciprocal(jnp.sum(e, axis=-1, keepdims=True),
                                   approx=False)


def _se_layer(x, w1, w2):
    b, c, h, w = x.shape
    hid, c_in = w1.shape
    out_ch, hid2 = w2.shape
    assert c_in == c and hid2 == hid

    hw = h * w
    x_flat = x.reshape(b, c, hw)

    tb = 8 if (b % 8 == 0 and b > 8) else b
    nb = b // tb

    w1t = jnp.asarray(w1).T * (1.0 / hw)
    w2t = jnp.asarray(w2).T

    block_bytes = tb * c * hw * 4
    vmem_limit = min(2 * block_bytes + (4 << 20), 56 << 20)

    out = pl.pallas_call(
        _se_body,
        out_shape=jax.ShapeDtypeStruct((b, out_ch), jnp.float32),
        grid=(nb,),
        in_specs=[
            pl.BlockSpec((tb, c, hw), lambda i: (i, 0, 0)),
            pl.BlockSpec((c, hid), lambda i: (0, 0)),
            pl.BlockSpec((hid, out_ch), lambda i: (0, 0)),
        ],
        out_specs=pl.BlockSpec((tb, out_ch), lambda i: (i, 0)),
        compiler_params=pltpu.CompilerParams(
            dimension_semantics=("parallel",),
            vmem_limit_bytes=vmem_limit,
        ),
    )(x_flat, w1t, w2t)

    return out.reshape(b, out_ch, 1, 1)


def kernel(x, w1, w2):
    return _se_layer(x, w1, w2)
